# Initial kernel scaffold; baseline (speedup 1.0000x reference)
import jax
import jax.numpy as jnp
from jax.experimental import pallas as pl


def kernel(w1, b1, wt, bt, w2, b2, x):
    raise NotImplementedError("write your pallas kernel here")


# speedup vs baseline: 3.0678x; 3.0678x over previous
import jax
import jax.numpy as jnp
from jax.experimental import pallas as pl
from jax.experimental.pallas import tpu as pltpu

_CIN = 128
_MID = 32
_COUT = 64
_H = 32
_W = 32
_HW = _H * _W


def _fused_body(x_ref, w1_ref, b1_ref, wt_ref, bt_ref, w2_ref, b2_ref, o_ref):
    xb = x_ref[0]

    dn = (((0,), (0,)), ((), ()))
    a = jax.lax.dot_general(w1_ref[...], xb, dn,
                            preferred_element_type=jnp.float32)
    a = jnp.maximum(a + b1_ref[...], 0.0).astype(jnp.bfloat16)

    taps = jax.lax.dot_general(wt_ref[...], a, dn,
                               preferred_element_type=jnp.float32)

    def tap(t):
        return taps[t * _MID:(t + 1) * _MID, :]

    lane = jax.lax.broadcasted_iota(jnp.int32, (_MID, _HW), 1)
    not_last_col = (lane & (_W - 1)) != (_W - 1)
    zcol = jnp.zeros((_MID, 1), jnp.float32)
    zrow = jnp.zeros((_MID, _W), jnp.float32)

    def shw(v):
        return jnp.where(not_last_col,
                         jnp.concatenate([v[:, 1:], zcol], axis=1), 0.0)

    def shh(v):
        return jnp.concatenate([v[:, _W:], zrow], axis=1)

    c00 = tap(4)
    c01 = tap(5) + shw(tap(3))
    c10 = tap(7) + shh(tap(1))
    c11 = tap(8) + shw(tap(6)) + shh(tap(2)) + shh(shw(tap(0)))

    bt = bt_ref[...]
    t_all = jnp.concatenate(
        [jnp.maximum(c + bt, 0.0).astype(jnp.bfloat16)
         for c in (c00, c01, c10, c11)], axis=1)
    out = jax.lax.dot_general(w2_ref[...], t_all, dn,
                              preferred_element_type=jnp.float32)
    out = jnp.maximum(out + b2_ref[...], 0.0)

    ox = jax.lax.broadcasted_iota(jnp.int32, (_W, 2 * _W), 1)
    k = jax.lax.broadcasted_iota(jnp.int32, (_W, 2 * _W), 0)
    e0 = (ox == 2 * k).astype(jnp.float32)
    e1 = (ox == 2 * k + 1).astype(jnp.float32)

    def p3d(i):
        return out[:, i * _HW:(i + 1) * _HW].reshape(_COUT, _H, _W)

    hi = jax.lax.Precision.HIGHEST
    dnx = (((2,), (0,)), ((), ()))

    def spread(p, e):
        return jax.lax.dot_general(p, e, dnx, precision=hi,
                                   preferred_element_type=jnp.float32)

    row_e = spread(p3d(0), e0) + spread(p3d(1), e1)
    row_o = spread(p3d(2), e0) + spread(p3d(3), e1)
    o_ref[0, :, 0::2, :] = row_e
    o_ref[0, :, 1::2, :] = row_o


def kernel(w1, b1, wt, bt, w2, b2, x):
    n = x.shape[0]
    x3 = x.reshape(n, _CIN, _HW).astype(jnp.bfloat16)
    b1c = b1.reshape(_MID, 1)
    btc = bt.reshape(_MID, 1)
    b2c = b2.reshape(_COUT, 1)

    out = pl.pallas_call(
        _fused_body,
        grid=(n,),
        in_specs=[
            pl.BlockSpec((1, _CIN, _HW), lambda i: (i, 0, 0)),
            pl.BlockSpec((_CIN, _MID), lambda i: (0, 0)),
            pl.BlockSpec((_MID, 1), lambda i: (0, 0)),
            pl.BlockSpec((_MID, 9 * _MID), lambda i: (0, 0)),
            pl.BlockSpec((_MID, 1), lambda i: (0, 0)),
            pl.BlockSpec((_MID, _COUT), lambda i: (0, 0)),
            pl.BlockSpec((_COUT, 1), lambda i: (0, 0)),
        ],
        out_specs=pl.BlockSpec((1, _COUT, 2 * _H, 2 * _W),
                               lambda i: (i, 0, 0, 0)),
        out_shape=jax.ShapeDtypeStruct((n, _COUT, 2 * _H, 2 * _W),
                                       jnp.float32),
        compiler_params=pltpu.CompilerParams(
            dimension_semantics=("parallel",),
            vmem_limit_bytes=100 * 1024 * 1024,
        ),
        cost_estimate=pl.CostEstimate(
            flops=2 * n * _HW * _MID * (_CIN + 9 * _MID + 4 * _COUT),
            transcendentals=0,
            bytes_accessed=n * (_CIN * _HW * 2 + _COUT * 4 * _HW * 4),
        ),
    )(x3, w1, b1c, wt, btc, w2, b2c)
    return out


# speedup vs baseline: 3.3542x; 1.0934x over previous
import jax
import jax.numpy as jnp
from jax.experimental import pallas as pl
from jax.experimental.pallas import tpu as pltpu

_CIN = 128
_MID = 32
_COUT = 64
_H = 32
_W = 32
_HW = _H * _W

_DN0 = (((0,), (0,)), ((), ()))
_DNX = (((2,), (0,)), ((), ()))


def _fused_body(x_ref, w1_ref, b1_ref, wt_ref, bt_ref, w2_ref, b2_ref, o_ref):
    xb = x_ref[0].reshape(_CIN, _HW).astype(jnp.bfloat16)

    a = jax.lax.dot_general(w1_ref[...], xb, _DN0,
                            preferred_element_type=jnp.float32)
    a = jnp.maximum(a + b1_ref[...], 0.0).astype(jnp.bfloat16)

    taps = jax.lax.dot_general(wt_ref[...], a, _DN0,
                               preferred_element_type=jnp.float32)
    taps = taps.astype(jnp.bfloat16).reshape(9 * _MID, _H, _W)

    def tap(t):
        return taps[t * _MID:(t + 1) * _MID].astype(jnp.float32)

    zc = jnp.zeros((_MID, _H, 1), jnp.float32)
    zr = jnp.zeros((_MID, 1, _W), jnp.float32)

    def shw(v):
        return jnp.concatenate([v[:, :, 1:], zc], axis=2)

    def shh(v):
        return jnp.concatenate([v[:, 1:, :], zr], axis=1)

    c00 = tap(4)
    c01 = tap(5) + shw(tap(3))
    c10 = tap(7) + shh(tap(1))
    c11 = tap(8) + shw(tap(6)) + shh(tap(2)) + shh(shw(tap(0)))

    bt = bt_ref[...][:, :, None]
    r00, r01, r10, r11 = (jnp.maximum(c + bt, 0.0).astype(jnp.bfloat16)
                          for c in (c00, c01, c10, c11))

    ox = jax.lax.broadcasted_iota(jnp.int32, (_W, 2 * _W), 1)
    kx = jax.lax.broadcasted_iota(jnp.int32, (_W, 2 * _W), 0)
    e0 = (ox == 2 * kx).astype(jnp.bfloat16)
    e1 = (ox == 2 * kx + 1).astype(jnp.bfloat16)

    def spread(p, e):
        return jax.lax.dot_general(
            p, e, _DNX, preferred_element_type=jnp.float32)

    t_e = (spread(r00, e0) + spread(r01, e1)).astype(jnp.bfloat16)
    t_o = (spread(r10, e0) + spread(r11, e1)).astype(jnp.bfloat16)

    b2 = b2_ref[...][:, :, None]
    for py, t in ((0, t_e), (1, t_o)):
        o = jax.lax.dot_general(w2_ref[...], t, _DN0,
                                preferred_element_type=jnp.float32)
        o_ref[0, :, py::2, :] = jnp.maximum(o + b2, 0.0)


def kernel(w1, b1, wt, bt, w2, b2, x):
    n = x.shape[0]
    b1c = b1.reshape(_MID, 1)
    btc = bt.reshape(_MID, 1)
    b2c = b2.reshape(_COUT, 1)

    out = pl.pallas_call(
        _fused_body,
        grid=(n,),
        in_specs=[
            pl.BlockSpec((1, _CIN, _H, _W), lambda i: (i, 0, 0, 0)),
            pl.BlockSpec((_CIN, _MID), lambda i: (0, 0)),
            pl.BlockSpec((_MID, 1), lambda i: (0, 0)),
            pl.BlockSpec((_MID, 9 * _MID), lambda i: (0, 0)),
            pl.BlockSpec((_MID, 1), lambda i: (0, 0)),
            pl.BlockSpec((_MID, _COUT), lambda i: (0, 0)),
            pl.BlockSpec((_COUT, 1), lambda i: (0, 0)),
        ],
        out_specs=pl.BlockSpec((1, _COUT, 2 * _H, 2 * _W),
                               lambda i: (i, 0, 0, 0)),
        out_shape=jax.ShapeDtypeStruct((n, _COUT, 2 * _H, 2 * _W),
                                       jnp.float32),
        compiler_params=pltpu.CompilerParams(
            dimension_semantics=("parallel",),
            vmem_limit_bytes=100 * 1024 * 1024,
        ),
        cost_estimate=pl.CostEstimate(
            flops=2 * n * _HW * _MID * (_CIN + 9 * _MID + 4 * _COUT),
            transcendentals=0,
            bytes_accessed=n * (_CIN * _HW * 4 + _COUT * 4 * _HW * 4),
        ),
    )(x, w1, b1c, wt, btc, w2, b2c)
    return out
